# Initial kernel scaffold; baseline (speedup 1.0000x reference)
import jax
import jax.numpy as jnp
from jax.experimental import pallas as pl


def kernel(patches, table):
    raise NotImplementedError("write your pallas kernel here")


# speedup vs baseline: 1.0131x; 1.0131x over previous
import jax
import jax.numpy as jnp
from jax.experimental import pallas as pl


def _add_kernel(p_ref, t_ref, o_ref):
    o_ref[...] = p_ref[...] + t_ref[...]


def kernel(patches, table):
    B, P, E = patches.shape
    return pl.pallas_call(
        _add_kernel,
        grid=(B,),
        in_specs=[
            pl.BlockSpec((1, P, E), lambda b: (b, 0, 0)),
            pl.BlockSpec((P, E), lambda b: (0, 0)),
        ],
        out_specs=pl.BlockSpec((1, P, E), lambda b: (b, 0, 0)),
        out_shape=jax.ShapeDtypeStruct((B, P, E), patches.dtype),
    )(patches, table)
